# baseline (speedup 1.0000x reference)
import functools

import jax
import jax.numpy as jnp
from jax import lax
from jax.experimental import pallas as pl
from jax.experimental.pallas import tpu as pltpu
from jax.experimental.pallas import tpu_sc as plsc

VOCAB = 100000
EMB_DIM = 64
BATCH = 1024
CTX = 10

NC, NS = 2, 16
NW = NC * NS
BPW = BATCH // NW
IPW = BPW * CTX
IPW_PAD = 384
NCHUNK = IPW_PAD // 128

VT = 2048
NV = (VOCAB + VT - 1) // VT
VTW = 2048
NVW = (VOCAB + VTW - 1) // VTW
TAIL = VOCAB - (NVW - 1) * VTW
TAIL_A = (TAIL // 128) * 128
TAIL_B = TAIL - TAIL_A
NBUF = 4
NEG = -1e30
LOG2E = 1.4426950408889634
LN2 = 0.6931471805599453


def _sc_gather_sum(xp, emb):
    mesh = plsc.VectorSubcoreMesh(core_axis_name="c", subcore_axis_name="s")

    @functools.partial(
        pl.kernel,
        mesh=mesh,
        compiler_params=pltpu.CompilerParams(use_tc_tiling_on_sc=False),
        out_type=jax.ShapeDtypeStruct((BATCH, EMB_DIM), jnp.float32),
        scratch_types=[
            pltpu.VMEM((NCHUNK, 128), jnp.int32),
            pltpu.VMEM((IPW_PAD, EMB_DIM), jnp.float32),
            pltpu.VMEM((BPW, EMB_DIM), jnp.float32),
            pltpu.SemaphoreType.DMA,
        ],
    )
    def k(xp_hbm, emb_hbm, out_hbm, idx_v, rows_v, acc_v, sem):
        wid = lax.axis_index("s") * NC + lax.axis_index("c")
        pltpu.sync_copy(xp_hbm.at[wid], idx_v)
        copies = [
            pltpu.async_copy(
                emb_hbm.at[idx_v.at[c]],
                rows_v.at[pl.ds(c * 128, 128)],
                sem,
            )
            for c in range(NCHUNK)
        ]
        for cp in copies:
            cp.wait()
        for bi in range(BPW):
            for c4 in range(EMB_DIM // 16):
                sl = pl.ds(c4 * 16, 16)
                acc = rows_v[bi * CTX, sl]
                for j in range(1, CTX):
                    acc = acc + rows_v[bi * CTX + j, sl]
                acc_v[bi, sl] = acc
        pltpu.sync_copy(acc_v, out_hbm.at[pl.ds(wid * BPW, BPW)])

    return k(xp, emb)


def _lse_body(s_ref, w_ref, b_ref, lse_ref, m_ref, l_ref):
    v = pl.program_id(0)
    w_aug = jnp.concatenate([w_ref[...], b_ref[...]], axis=1)
    t = lax.dot_general(
        s_ref[...].astype(jnp.bfloat16), w_aug.astype(jnp.bfloat16),
        (((1,), (1,)), ((), ())),
        preferred_element_type=jnp.float32,
    )
    col = lax.broadcasted_iota(jnp.int32, (1, VT), 1)
    t = jnp.where(col < (VOCAB - v * VT), t, NEG)

    @pl.when(v == 0)
    def _init():
        m_ref[...] = jnp.full((BATCH, 1), NEG, jnp.float32)
        l_ref[...] = jnp.zeros((BATCH, 1), jnp.float32)

    tmax = jnp.max(t, axis=1, keepdims=True)
    m_new = jnp.maximum(m_ref[...], tmax)
    l_ref[...] = (l_ref[...] * jnp.exp2(m_ref[...] - m_new)
                  + jnp.sum(jnp.exp2(t - m_new), axis=1, keepdims=True))
    m_ref[...] = m_new

    @pl.when(v == NV - 1)
    def _fin():
        lse_ref[...] = LN2 * (m_ref[...] + jnp.log2(l_ref[...]))


def _lse_pass(s_scaled, W, b2col):
    return pl.pallas_call(
        _lse_body,
        grid=(NV,),
        in_specs=[
            pl.BlockSpec((BATCH, EMB_DIM + 1), lambda v: (0, 0)),
            pl.BlockSpec((VT, EMB_DIM), lambda v: (v, 0)),
            pl.BlockSpec((VT, 1), lambda v: (v, 0)),
        ],
        out_specs=pl.BlockSpec((BATCH, 1), lambda v: (0, 0)),
        out_shape=jax.ShapeDtypeStruct((BATCH, 1), jnp.float32),
        scratch_shapes=[
            pltpu.VMEM((BATCH, 1), jnp.float32),
            pltpu.VMEM((BATCH, 1), jnp.float32),
        ],
    )(s_scaled, W, b2col)


def _write_body(s_ref, w_ref, b_ref, out_hbm, tail_ref, buf, sem):
    v = pl.program_id(0)
    slot = lax.rem(v, NBUF)

    @pl.when(v >= NBUF)
    def _wait_slot():
        pltpu.make_async_copy(
            buf.at[slot],
            out_hbm.at[:, pl.ds((v - NBUF) * VTW, VTW)],
            sem.at[slot],
        ).wait()

    w_aug = jnp.concatenate(
        [w_ref[...], b_ref[...], jnp.full((VTW, 1), -1.0, jnp.float32)], axis=1
    )
    tile = lax.dot_general(
        s_ref[...].astype(jnp.bfloat16), w_aug.astype(jnp.bfloat16),
        (((1,), (1,)), ((), ())),
        preferred_element_type=jnp.float32,
    )
    for k in range(NBUF):
        @pl.when(slot == k)
        def _store(k=k):
            buf[k] = tile

    @pl.when(v < NVW - 1)
    def _start_full():
        pltpu.make_async_copy(
            buf.at[slot],
            out_hbm.at[:, pl.ds(v * VTW, VTW)],
            sem.at[slot],
        ).start()

    @pl.when(v == NVW - 1)
    def _start_tail_and_drain():
        tail_ref[...] = tile[:, TAIL_A:TAIL_A + TAIL_B]
        pltpu.make_async_copy(
            buf.at[slot, :, pl.ds(0, TAIL_A)],
            out_hbm.at[:, pl.ds((NVW - 1) * VTW, TAIL_A)],
            sem.at[slot],
        ).start()
        for k in range(NBUF):
            vv = NVW - NBUF + k
            sl = vv % NBUF
            if vv < NVW - 1:
                pltpu.make_async_copy(
                    buf.at[sl],
                    out_hbm.at[:, pl.ds(vv * VTW, VTW)],
                    sem.at[sl],
                ).wait()
            else:
                pltpu.make_async_copy(
                    buf.at[sl, :, pl.ds(0, TAIL_A)],
                    out_hbm.at[:, pl.ds(vv * VTW, TAIL_A)],
                    sem.at[sl],
                ).wait()


def _write_pass(s_aug, W, b2col):
    return pl.pallas_call(
        _write_body,
        grid=(NVW,),
        in_specs=[
            pl.BlockSpec((BATCH, EMB_DIM + 2), lambda v: (0, 0)),
            pl.BlockSpec((VTW, EMB_DIM), lambda v: (v, 0)),
            pl.BlockSpec((VTW, 1), lambda v: (v, 0)),
        ],
        out_specs=[
            pl.BlockSpec(memory_space=pl.ANY),
            pl.BlockSpec((BATCH, TAIL_B), lambda v: (0, 0)),
        ],
        out_shape=[
            jax.ShapeDtypeStruct((BATCH, VOCAB), jnp.float32),
            jax.ShapeDtypeStruct((BATCH, TAIL_B), jnp.float32),
        ],
        scratch_shapes=[
            pltpu.VMEM((NBUF, BATCH, VTW), jnp.float32),
            pltpu.SemaphoreType.DMA((NBUF,)),
        ],
    )(s_aug, W, b2col)


def kernel(x, emb, W, b):
    xf = x.astype(jnp.int32).reshape(NW, IPW)
    xp = jnp.pad(xf, ((0, 0), (0, IPW_PAD - IPW))).reshape(NW, NCHUNK, 128)
    s = _sc_gather_sum(xp, emb)
    ones = jnp.ones((BATCH, 1), jnp.float32)
    b2col = b.reshape(VOCAB, 1)
    s_scaled = jnp.concatenate([s * LOG2E, ones], axis=1)
    lse = _lse_pass(s_scaled, W, b2col * LOG2E)
    s_aug = jnp.concatenate([s, ones, lse], axis=1)
    out_main, tail = _write_pass(s_aug, W, b2col)
    return lax.dynamic_update_slice(out_main, tail, (0, VOCAB - TAIL_B))


# speedup vs baseline: 1.0038x; 1.0038x over previous
import functools

import jax
import jax.numpy as jnp
from jax import lax
from jax.experimental import pallas as pl
from jax.experimental.pallas import tpu as pltpu
from jax.experimental.pallas import tpu_sc as plsc

VOCAB = 100000
EMB_DIM = 64
BATCH = 1024
CTX = 10

NC, NS = 2, 16
NW = NC * NS
BPW = BATCH // NW
IPW = BPW * CTX
IPW_PAD = 384
NCHUNK = IPW_PAD // 128

VT = 2048
NV = (VOCAB + VT - 1) // VT
VTW = 4096
NVW = (VOCAB + VTW - 1) // VTW
TAIL = VOCAB - (NVW - 1) * VTW
TAIL_A = (TAIL // 128) * 128
TAIL_B = TAIL - TAIL_A
NBUF = 2
NEG = -1e30
LOG2E = 1.4426950408889634
LN2 = 0.6931471805599453


def _sc_gather_sum(xp, emb):
    mesh = plsc.VectorSubcoreMesh(core_axis_name="c", subcore_axis_name="s")

    @functools.partial(
        pl.kernel,
        mesh=mesh,
        compiler_params=pltpu.CompilerParams(use_tc_tiling_on_sc=False),
        out_type=jax.ShapeDtypeStruct((BATCH, EMB_DIM), jnp.float32),
        scratch_types=[
            pltpu.VMEM((NCHUNK, 128), jnp.int32),
            pltpu.VMEM((IPW_PAD, EMB_DIM), jnp.float32),
            pltpu.VMEM((BPW, EMB_DIM), jnp.float32),
            pltpu.SemaphoreType.DMA,
        ],
    )
    def k(xp_hbm, emb_hbm, out_hbm, idx_v, rows_v, acc_v, sem):
        wid = lax.axis_index("s") * NC + lax.axis_index("c")
        pltpu.sync_copy(xp_hbm.at[wid], idx_v)
        copies = [
            pltpu.async_copy(
                emb_hbm.at[idx_v.at[c]],
                rows_v.at[pl.ds(c * 128, 128)],
                sem,
            )
            for c in range(NCHUNK)
        ]
        for cp in copies:
            cp.wait()
        for bi in range(BPW):
            for c4 in range(EMB_DIM // 16):
                sl = pl.ds(c4 * 16, 16)
                acc = rows_v[bi * CTX, sl]
                for j in range(1, CTX):
                    acc = acc + rows_v[bi * CTX + j, sl]
                acc_v[bi, sl] = acc
        pltpu.sync_copy(acc_v, out_hbm.at[pl.ds(wid * BPW, BPW)])

    return k(xp, emb)


def _lse_body(s_ref, w_ref, b_ref, lse_ref, m_ref, l_ref):
    v = pl.program_id(0)
    w_aug = jnp.concatenate([w_ref[...], b_ref[...]], axis=1)
    t = lax.dot_general(
        s_ref[...].astype(jnp.bfloat16), w_aug.astype(jnp.bfloat16),
        (((1,), (1,)), ((), ())),
        preferred_element_type=jnp.float32,
    )
    col = lax.broadcasted_iota(jnp.int32, (1, VT), 1)
    t = jnp.where(col < (VOCAB - v * VT), t, NEG)

    @pl.when(v == 0)
    def _init():
        m_ref[...] = jnp.full((BATCH, 1), NEG, jnp.float32)
        l_ref[...] = jnp.zeros((BATCH, 1), jnp.float32)

    tmax = jnp.max(t, axis=1, keepdims=True)
    m_new = jnp.maximum(m_ref[...], tmax)
    l_ref[...] = (l_ref[...] * jnp.exp2(m_ref[...] - m_new)
                  + jnp.sum(jnp.exp2(t - m_new), axis=1, keepdims=True))
    m_ref[...] = m_new

    @pl.when(v == NV - 1)
    def _fin():
        lse_ref[...] = LN2 * (m_ref[...] + jnp.log2(l_ref[...]))


def _lse_pass(s_scaled, W, b2col):
    return pl.pallas_call(
        _lse_body,
        grid=(NV,),
        in_specs=[
            pl.BlockSpec((BATCH, EMB_DIM + 1), lambda v: (0, 0)),
            pl.BlockSpec((VT, EMB_DIM), lambda v: (v, 0)),
            pl.BlockSpec((VT, 1), lambda v: (v, 0)),
        ],
        out_specs=pl.BlockSpec((BATCH, 1), lambda v: (0, 0)),
        out_shape=jax.ShapeDtypeStruct((BATCH, 1), jnp.float32),
        scratch_shapes=[
            pltpu.VMEM((BATCH, 1), jnp.float32),
            pltpu.VMEM((BATCH, 1), jnp.float32),
        ],
    )(s_scaled, W, b2col)


def _write_body(s_ref, w_ref, b_ref, out_hbm, tail_ref, buf, sem):
    v = pl.program_id(0)
    slot = lax.rem(v, NBUF)

    @pl.when(v >= NBUF)
    def _wait_slot():
        pltpu.make_async_copy(
            buf.at[slot],
            out_hbm.at[:, pl.ds((v - NBUF) * VTW, VTW)],
            sem.at[slot],
        ).wait()

    w_aug = jnp.concatenate(
        [w_ref[...], b_ref[...], jnp.full((VTW, 1), -1.0, jnp.float32)], axis=1
    )
    tile = lax.dot_general(
        s_ref[...].astype(jnp.bfloat16), w_aug.astype(jnp.bfloat16),
        (((1,), (1,)), ((), ())),
        preferred_element_type=jnp.float32,
    )
    for k in range(NBUF):
        @pl.when(slot == k)
        def _store(k=k):
            buf[k] = tile

    @pl.when(v < NVW - 1)
    def _start_full():
        pltpu.make_async_copy(
            buf.at[slot],
            out_hbm.at[:, pl.ds(v * VTW, VTW)],
            sem.at[slot],
        ).start()

    @pl.when(v == NVW - 1)
    def _start_tail_and_drain():
        tail_ref[...] = tile[:, TAIL_A:TAIL_A + TAIL_B]
        pltpu.make_async_copy(
            buf.at[slot, :, pl.ds(0, TAIL_A)],
            out_hbm.at[:, pl.ds((NVW - 1) * VTW, TAIL_A)],
            sem.at[slot],
        ).start()
        for k in range(NBUF):
            vv = NVW - NBUF + k
            sl = vv % NBUF
            if vv < NVW - 1:
                pltpu.make_async_copy(
                    buf.at[sl],
                    out_hbm.at[:, pl.ds(vv * VTW, VTW)],
                    sem.at[sl],
                ).wait()
            else:
                pltpu.make_async_copy(
                    buf.at[sl, :, pl.ds(0, TAIL_A)],
                    out_hbm.at[:, pl.ds(vv * VTW, TAIL_A)],
                    sem.at[sl],
                ).wait()


def _write_pass(s_aug, W, b2col):
    return pl.pallas_call(
        _write_body,
        grid=(NVW,),
        in_specs=[
            pl.BlockSpec((BATCH, EMB_DIM + 2), lambda v: (0, 0)),
            pl.BlockSpec((VTW, EMB_DIM), lambda v: (v, 0)),
            pl.BlockSpec((VTW, 1), lambda v: (v, 0)),
        ],
        out_specs=[
            pl.BlockSpec(memory_space=pl.ANY),
            pl.BlockSpec((BATCH, TAIL_B), lambda v: (0, 0)),
        ],
        out_shape=[
            jax.ShapeDtypeStruct((BATCH, VOCAB), jnp.float32),
            jax.ShapeDtypeStruct((BATCH, TAIL_B), jnp.float32),
        ],
        scratch_shapes=[
            pltpu.VMEM((NBUF, BATCH, VTW), jnp.float32),
            pltpu.SemaphoreType.DMA((NBUF,)),
        ],
        compiler_params=pltpu.CompilerParams(
            vmem_limit_bytes=62 * 1024 * 1024,
        ),
    )(s_aug, W, b2col)


def kernel(x, emb, W, b):
    xf = x.astype(jnp.int32).reshape(NW, IPW)
    xp = jnp.pad(xf, ((0, 0), (0, IPW_PAD - IPW))).reshape(NW, NCHUNK, 128)
    s = _sc_gather_sum(xp, emb)
    ones = jnp.ones((BATCH, 1), jnp.float32)
    b2col = b.reshape(VOCAB, 1)
    s_scaled = jnp.concatenate([s * LOG2E, ones], axis=1)
    lse = _lse_pass(s_scaled, W, b2col * LOG2E)
    s_aug = jnp.concatenate([s, ones, lse], axis=1)
    out_main, tail = _write_pass(s_aug, W, b2col)
    return lax.dynamic_update_slice(out_main, tail, (0, VOCAB - TAIL_B))
